# baseline (speedup 1.0000x reference)
import functools
import math

import jax
import jax.numpy as jnp
from jax import lax
from jax.experimental import pallas as pl
from jax.experimental.pallas import tpu as pltpu
from jax.experimental.pallas import tpu_sc as plsc

D_MODEL = 64
SCALE = math.sqrt(D_MODEL)

NUM_CORES = 2
NUM_SUBCORES = 16
NUM_WORKERS = NUM_CORES * NUM_SUBCORES
LANES = 16
SSTRIDE = LANES + 1


def _emb_kernel(b0_dim, b1_dim):
    blk = b0_dim // NUM_WORKERS
    assert blk % LANES == 0
    n_blk = blk // LANES
    n_dblk = D_MODEL // LANES
    mesh = plsc.VectorSubcoreMesh(core_axis_name="c", subcore_axis_name="s")

    @functools.partial(
        pl.kernel,
        mesh=mesh,
        out_type=jax.ShapeDtypeStruct((b1_dim, D_MODEL, b0_dim), jnp.float32),
        scratch_types=[
            pltpu.VMEM((b1_dim, blk), jnp.int32),
            pltpu.VMEM((blk, 2 * D_MODEL), jnp.float32),
            pltpu.VMEM((blk, 2 * D_MODEL), jnp.float32),
            pltpu.VMEM((D_MODEL, blk), jnp.float32),
            pltpu.VMEM((D_MODEL, blk), jnp.float32),
            pltpu.VMEM((LANES * SSTRIDE,), jnp.float32),
            pltpu.SemaphoreType.DMA,
            pltpu.SemaphoreType.DMA,
            pltpu.SemaphoreType.DMA,
            pltpu.SemaphoreType.DMA,
        ],
        compiler_params=pltpu.CompilerParams(
            use_tc_tiling_on_sc=True, needs_layout_passes=False
        ),
    )
    def k(xt_hbm, table_hbm, out_hbm, idx_v, g0, g1, t0, t1, scr,
          gs0, gs1, ss0, ss1):
        cid = lax.axis_index("c")
        sid = lax.axis_index("s")
        wid = sid * NUM_CORES + cid
        col0 = wid * blk

        pltpu.sync_copy(xt_hbm.at[:, pl.ds(col0, blk)], idx_v)

        def gather(b1, g, sem):
            b1c = jnp.minimum(b1, b1_dim - 1)
            return pltpu.make_async_copy(
                table_hbm.at[idx_v.at[b1c]], g, sem
            )

        def store(b1, t, sem):
            return pltpu.make_async_copy(
                t, out_hbm.at[b1, :, pl.ds(col0, blk)], sem
            )

        iota = lax.iota(jnp.int32, LANES)
        scatsel = iota * SSTRIDE

        def transpose_scale(g, t):
            def blk_body(it, carry):
                r0 = (it // n_dblk) * LANES
                c0 = (it % n_dblk) * LANES
                for r in range(LANES):
                    v = g[r0 + r, pl.ds(c0, LANES)]
                    plsc.store_scatter(scr, [scatsel + r], v)
                for c in range(LANES):
                    w = scr[pl.ds(c * SSTRIDE, LANES)]
                    t[c0 + c, pl.ds(r0, LANES)] = w * SCALE
                return carry

            lax.fori_loop(0, n_blk * n_dblk, blk_body, 0)

        gather(0, g0, gs0).start()
        gather(1, g1, gs1).start()

        def body(j, carry):
            b1 = j * 2
            gather(b1, g0, gs0).wait()
            transpose_scale(g0, t0)
            store(b1, t0, ss0).start()
            gather(b1 + 2, g0, gs0).start()
            gather(b1 + 1, g1, gs1).wait()
            transpose_scale(g1, t1)
            store(b1 + 1, t1, ss1).start()
            gather(b1 + 3, g1, gs1).start()
            store(b1, t0, ss0).wait()
            store(b1 + 1, t1, ss1).wait()
            return carry

        lax.fori_loop(0, b1_dim // 2, body, 0)

        gather(b1_dim - 1, g0, gs0).wait()
        gather(b1_dim - 1, g1, gs1).wait()

    return k


def kernel(x, table):
    b0, b1 = x.shape
    xt = jnp.swapaxes(x, 0, 1).astype(jnp.int32)
    tpad = jnp.pad(table, ((0, 0), (0, D_MODEL)))
    out = _emb_kernel(b0, b1)(xt, tpad)
    return jnp.transpose(out, (2, 0, 1))
